# baseline (speedup 1.0000x reference)
import functools

import jax
from jax import lax
import jax.numpy as jnp
from jax.experimental import pallas as pl
import jax.experimental.pallas.tpu as pltpu
from jax.experimental.pallas import tpu_sc as plsc

HIDDEN = 1024
E = 8
T_TOTAL = 8192
TB = 512
NUM_TB = T_TOTAL // TB

NC = 2
NS = 16
NW = NC * NS
ROWS_PER_W = T_TOTAL // NW
ZROWS = 32
NCHUNK = ROWS_PER_W // ZROWS


def _dot_t(a, b):
    return jax.lax.dot_general(
        a, b, (((1,), (1,)), ((), ())), preferred_element_type=jnp.float32
    )


def _router_expert_kernel(x_ref, x8_ref, gw_ref, gb_ref, ew_ref, eb_ref,
                          out8_ref, logits_ref, acc_ref):
    t = pl.program_id(0)

    xb = x_ref[:, :]
    gw = gw_ref[:, :]
    gb = gb_ref[:, :]
    logits_ref[:, :] = _dot_t(xb, gw) + gb

    @pl.when(t == 0)
    def _init():
        acc_ref[:, :, :] = jnp.zeros((2, E, HIDDEN // 2), jnp.float32)

    x8 = x8_ref[:, :]
    l8 = _dot_t(x8, gw) + gb
    m = jnp.max(l8, axis=-1, keepdims=True)
    p = jnp.exp(l8 - m)
    p = p / jnp.sum(p, axis=-1, keepdims=True)

    iota_e = jax.lax.broadcasted_iota(jnp.int32, (E, E), 1).astype(jnp.float32)
    iota_r = jax.lax.broadcasted_iota(jnp.int32, (E, E), 0).astype(jnp.float32)

    rank = jnp.zeros((E, E), jnp.float32)
    for e2 in range(E):
        pe2 = p[:, e2:e2 + 1]
        rank = rank + jnp.where(
            (pe2 > p) | ((pe2 == p) & (e2 < iota_e)), 1.0, 0.0)

    sp = jnp.zeros((E, E), jnp.float32)
    for e in range(E):
        sp = sp + jnp.where(rank[:, e:e + 1] == iota_e,
                            p[:, e:e + 1], 0.0)
    sp = sp / jnp.sum(sp, axis=-1, keepdims=True)

    i = t // 2
    h = t % 2
    fi = i.astype(jnp.float32)
    rank_i = jnp.sum(jnp.where(iota_r == fi, rank, 0.0),
                     axis=0, keepdims=True)
    c = sp * jnp.where(rank_i == iota_r, 1.0, 0.0)

    y = _dot_t(x8, ew_ref[0]) + eb_ref[0]

    acc_ref[h] += jax.lax.dot_general(
        c, y, (((1,), (0,)), ((), ())),
        preferred_element_type=jnp.float32)

    @pl.when(t == NUM_TB - 1)
    def _final():
        out8_ref[:, 0:HIDDEN // 2] = acc_ref[0]
        out8_ref[:, HIDDEN // 2:HIDDEN] = acc_ref[1]


def _zero_fill_kernel(out_hbm, zbuf, sem):
    cid = lax.axis_index("c")
    sid = lax.axis_index("s")
    wid = sid * NC + cid

    def zero_row(r, _):
        def zero_vec(j, _):
            zbuf[r, pl.ds(j * 16, 16)] = jnp.zeros((16,), jnp.float32)
            return 0
        return lax.fori_loop(0, HIDDEN // 16, zero_vec, 0, unroll=8)

    lax.fori_loop(0, ZROWS, zero_row, 0)

    base = wid * ROWS_PER_W
    copies = [
        pltpu.async_copy(zbuf, out_hbm.at[pl.ds(base + k * ZROWS, ZROWS)], sem)
        for k in range(NCHUNK)
    ]
    for cp in copies:
        cp.wait()


def _patch_kernel(z_ref, o8_ref, out_ref):
    del z_ref
    out_ref[:, :] = o8_ref[:, :]


@jax.jit
def kernel(x, gate_W, gate_b, expert_W, expert_b):
    B, S, H = x.shape
    xf = x.reshape(B * S, H)
    gb2 = gate_b.reshape(1, E)
    ew_half = expert_W.reshape(2 * E, H // 2, H)
    eb_half = expert_b.reshape(2 * E, 1, H // 2)

    out8, logits = pl.pallas_call(
        _router_expert_kernel,
        grid=(NUM_TB,),
        in_specs=[
            pl.BlockSpec((TB, H), lambda i: (i, 0)),
            pl.BlockSpec((E, H), lambda i: (0, 0)),
            pl.BlockSpec((E, H), lambda i: (0, 0)),
            pl.BlockSpec((1, E), lambda i: (0, 0)),
            pl.BlockSpec((1, H // 2, H), lambda i: (i, 0, 0)),
            pl.BlockSpec((1, 1, H // 2), lambda i: (i, 0, 0)),
        ],
        out_specs=[
            pl.BlockSpec((E, H), lambda i: (0, 0)),
            pl.BlockSpec((TB, E), lambda i: (i, 0)),
        ],
        out_shape=[
            jax.ShapeDtypeStruct((E, H), jnp.float32),
            jax.ShapeDtypeStruct((B * S, E), jnp.float32),
        ],
        scratch_shapes=[pltpu.VMEM((2, E, H // 2), jnp.float32)],
    )(xf, xf, gate_W, gb2, ew_half, eb_half)

    zero_fill = functools.partial(
        pl.kernel,
        mesh=plsc.VectorSubcoreMesh(core_axis_name="c", subcore_axis_name="s"),
        out_type=jax.ShapeDtypeStruct((B * S, H), jnp.float32),
        scratch_types=[
            pltpu.VMEM((ZROWS, H), jnp.float32),
            pltpu.SemaphoreType.DMA,
        ],
    )(_zero_fill_kernel)
    out_z = zero_fill()

    out = pl.pallas_call(
        _patch_kernel,
        grid=(1,),
        in_specs=[
            pl.BlockSpec((E, H), lambda i: (0, 0)),
            pl.BlockSpec((E, H), lambda i: (0, 0)),
        ],
        out_specs=pl.BlockSpec((E, H), lambda i: (0, 0)),
        out_shape=jax.ShapeDtypeStruct((B * S, H), jnp.float32),
        input_output_aliases={0: 0},
    )(out_z, out8)

    return out.reshape(B, S, H), logits


# speedup vs baseline: 1.1178x; 1.1178x over previous
import functools

import jax
from jax import lax
import jax.numpy as jnp
from jax.experimental import pallas as pl
import jax.experimental.pallas.tpu as pltpu
from jax.experimental.pallas import tpu_sc as plsc

HIDDEN = 1024
E = 8
T_TOTAL = 8192
TB = 512
NUM_TB = T_TOTAL // TB

NC = 2
NS = 16
NW = NC * NS
ROWS_PER_W = T_TOTAL // NW
ZROWS = 32
NCHUNK = ROWS_PER_W // ZROWS


def _dot_t(a, b):
    return jax.lax.dot_general(
        a, b, (((1,), (1,)), ((), ())), preferred_element_type=jnp.float32
    )


def _router_expert_kernel(x_ref, x8_ref, gw_ref, gb_ref, gbt_ref, ew_ref,
                          eb_ref, out8_ref, logits_ref, acc_ref):
    t = pl.program_id(0)

    xb = x_ref[:, :]
    gw = gw_ref[:, :]
    gb = gb_ref[:, :]
    logits_ref[:, :] = _dot_t(gw, xb) + gbt_ref[:, :]

    @pl.when(t == 0)
    def _init():
        acc_ref[:, :, :] = jnp.zeros((2, E, HIDDEN // 2), jnp.float32)

    x8 = x8_ref[:, :]
    l8 = _dot_t(x8, gw) + gb
    m = jnp.max(l8, axis=-1, keepdims=True)
    p = jnp.exp(l8 - m)
    p = p / jnp.sum(p, axis=-1, keepdims=True)

    iota_e = jax.lax.broadcasted_iota(jnp.int32, (E, E), 1).astype(jnp.float32)
    iota_r = jax.lax.broadcasted_iota(jnp.int32, (E, E), 0).astype(jnp.float32)

    rank = jnp.zeros((E, E), jnp.float32)
    for e2 in range(E):
        pe2 = p[:, e2:e2 + 1]
        rank = rank + jnp.where(
            (pe2 > p) | ((pe2 == p) & (e2 < iota_e)), 1.0, 0.0)

    sp = jnp.zeros((E, E), jnp.float32)
    for e in range(E):
        sp = sp + jnp.where(rank[:, e:e + 1] == iota_e,
                            p[:, e:e + 1], 0.0)
    sp = sp / jnp.sum(sp, axis=-1, keepdims=True)

    i = t // 2
    h = t % 2
    fi = i.astype(jnp.float32)
    rank_i = jnp.sum(jnp.where(iota_r == fi, rank, 0.0),
                     axis=0, keepdims=True)
    c = sp * jnp.where(rank_i == iota_r, 1.0, 0.0)

    y = _dot_t(x8, ew_ref[0]) + eb_ref[0]

    acc_ref[h] += jax.lax.dot_general(
        c, y, (((1,), (0,)), ((), ())),
        preferred_element_type=jnp.float32)

    @pl.when(t == NUM_TB - 1)
    def _final():
        out8_ref[:, 0:HIDDEN // 2] = acc_ref[0]
        out8_ref[:, HIDDEN // 2:HIDDEN] = acc_ref[1]


def _zero_fill_kernel(out_hbm, zbuf, sem):
    cid = lax.axis_index("c")
    sid = lax.axis_index("s")
    wid = sid * NC + cid

    def zero_row(r, _):
        def zero_vec(j, _):
            zbuf[r, pl.ds(j * 16, 16)] = jnp.zeros((16,), jnp.float32)
            return 0
        return lax.fori_loop(0, HIDDEN // 16, zero_vec, 0, unroll=8)

    lax.fori_loop(0, ZROWS, zero_row, 0)

    base = wid * ROWS_PER_W
    copies = [
        pltpu.async_copy(zbuf, out_hbm.at[pl.ds(base + k * ZROWS, ZROWS)], sem)
        for k in range(NCHUNK)
    ]
    for cp in copies:
        cp.wait()


def _patch_kernel(z_ref, o8_ref, out_ref):
    del z_ref
    out_ref[:, :] = o8_ref[:, :]


@jax.jit
def kernel(x, gate_W, gate_b, expert_W, expert_b):
    B, S, H = x.shape
    xf = x.reshape(B * S, H)
    gb2 = gate_b.reshape(1, E)
    ew_half = expert_W.reshape(2 * E, H // 2, H)
    eb_half = expert_b.reshape(2 * E, 1, H // 2)

    out8, logits = pl.pallas_call(
        _router_expert_kernel,
        grid=(NUM_TB,),
        in_specs=[
            pl.BlockSpec((TB, H), lambda i: (i, 0)),
            pl.BlockSpec((E, H), lambda i: (0, 0)),
            pl.BlockSpec((E, H), lambda i: (0, 0)),
            pl.BlockSpec((1, E), lambda i: (0, 0)),
            pl.BlockSpec((E, 1), lambda i: (0, 0)),
            pl.BlockSpec((1, H // 2, H), lambda i: (i, 0, 0)),
            pl.BlockSpec((1, 1, H // 2), lambda i: (i, 0, 0)),
        ],
        out_specs=[
            pl.BlockSpec((E, H), lambda i: (0, 0)),
            pl.BlockSpec((E, TB), lambda i: (0, i)),
        ],
        out_shape=[
            jax.ShapeDtypeStruct((E, H), jnp.float32),
            jax.ShapeDtypeStruct((E, B * S), jnp.float32),
        ],
        scratch_shapes=[pltpu.VMEM((2, E, H // 2), jnp.float32)],
    )(xf, xf, gate_W, gb2, gate_b.reshape(E, 1), ew_half, eb_half)

    zero_fill = functools.partial(
        pl.kernel,
        mesh=plsc.VectorSubcoreMesh(core_axis_name="c", subcore_axis_name="s"),
        out_type=jax.ShapeDtypeStruct((B * S, H), jnp.float32),
        scratch_types=[
            pltpu.VMEM((ZROWS, H), jnp.float32),
            pltpu.SemaphoreType.DMA,
        ],
    )(_zero_fill_kernel)
    out_z = zero_fill()

    out = pl.pallas_call(
        _patch_kernel,
        grid=(1,),
        in_specs=[
            pl.BlockSpec((E, H), lambda i: (0, 0)),
            pl.BlockSpec((E, H), lambda i: (0, 0)),
        ],
        out_specs=pl.BlockSpec((E, H), lambda i: (0, 0)),
        out_shape=jax.ShapeDtypeStruct((B * S, H), jnp.float32),
        input_output_aliases={0: 0},
    )(out_z, out8)

    return out.reshape(B, S, H), logits.T


# speedup vs baseline: 1.1953x; 1.0694x over previous
import functools

import jax
from jax import lax
import jax.numpy as jnp
from jax.experimental import pallas as pl
import jax.experimental.pallas.tpu as pltpu
from jax.experimental.pallas import tpu_sc as plsc

HIDDEN = 1024
E = 8
T_TOTAL = 8192
TB = 1024
NUM_TB = T_TOTAL // TB

NC = 2
NS = 16
NW = NC * NS
ROWS_PER_W = T_TOTAL // NW
ZROWS = 32
NCHUNK = ROWS_PER_W // ZROWS


def _dot_t(a, b):
    return jax.lax.dot_general(
        a, b, (((1,), (1,)), ((), ())), preferred_element_type=jnp.float32
    )


def _router_expert_kernel(x_ref, x8_ref, gw_ref, gb_ref, gbt_ref, ew_ref,
                          eb_ref, out8_ref, logits_ref, c_ref):
    t = pl.program_id(0)

    xb = x_ref[:, :]
    gw = gw_ref[:, :]
    gb = gb_ref[:, :]
    logits_ref[:, :] = _dot_t(gw, xb) + gbt_ref[:, :]

    x8 = x8_ref[:, :]

    @pl.when(t == 0)
    def _init():
        l8 = _dot_t(x8, gw) + gb
        m = jnp.max(l8, axis=-1, keepdims=True)
        p = jnp.exp(l8 - m)
        p = p / jnp.sum(p, axis=-1, keepdims=True)

        iota_e = jax.lax.broadcasted_iota(
            jnp.int32, (E, E), 1).astype(jnp.float32)
        iota_r = jax.lax.broadcasted_iota(
            jnp.int32, (E, E), 0).astype(jnp.float32)

        rank = jnp.zeros((E, E), jnp.float32)
        for e2 in range(E):
            pe2 = p[:, e2:e2 + 1]
            rank = rank + jnp.where(
                (pe2 > p) | ((pe2 == p) & (e2 < iota_e)), 1.0, 0.0)

        sp = jnp.zeros((E, E), jnp.float32)
        for e in range(E):
            sp = sp + jnp.where(rank[:, e:e + 1] == iota_e,
                                p[:, e:e + 1], 0.0)
        sp = sp / jnp.sum(sp, axis=-1, keepdims=True)

        for i in range(E):
            ri = rank[i:i + 1, :]
            c_ref[i] = sp * jnp.where(ri == iota_r, 1.0, 0.0)

        out8_ref[:, :] = jnp.zeros((E, HIDDEN), jnp.float32)

    y = _dot_t(x8, ew_ref[0]) + eb_ref[0]

    out8_ref[:, :] += jax.lax.dot_general(
        c_ref[t], y, (((1,), (0,)), ((), ())),
        preferred_element_type=jnp.float32)


def _zero_fill_kernel(out_hbm, zbuf, sem):
    cid = lax.axis_index("c")
    sid = lax.axis_index("s")
    wid = sid * NC + cid

    def zero_row(r, _):
        def zero_vec(j, _):
            zbuf[r, pl.ds(j * 16, 16)] = jnp.zeros((16,), jnp.float32)
            return 0
        return lax.fori_loop(0, HIDDEN // 16, zero_vec, 0, unroll=8)

    lax.fori_loop(0, ZROWS, zero_row, 0)

    base = wid * ROWS_PER_W
    copies = [
        pltpu.async_copy(zbuf, out_hbm.at[pl.ds(base + k * ZROWS, ZROWS)], sem)
        for k in range(NCHUNK)
    ]
    for cp in copies:
        cp.wait()


def _patch_kernel(z_ref, o8_ref, out_ref):
    del z_ref
    out_ref[:, :] = o8_ref[:, :]


@jax.jit
def kernel(x, gate_W, gate_b, expert_W, expert_b):
    B, S, H = x.shape
    xf = x.reshape(B * S, H)
    gb2 = gate_b.reshape(1, E)
    ew_half = expert_W
    eb_half = expert_b.reshape(E, 1, H)

    out8, logits = pl.pallas_call(
        _router_expert_kernel,
        grid=(NUM_TB,),
        in_specs=[
            pl.BlockSpec((TB, H), lambda i: (i, 0)),
            pl.BlockSpec((E, H), lambda i: (0, 0)),
            pl.BlockSpec((E, H), lambda i: (0, 0)),
            pl.BlockSpec((1, E), lambda i: (0, 0)),
            pl.BlockSpec((E, 1), lambda i: (0, 0)),
            pl.BlockSpec((1, H, H), lambda i: (i, 0, 0)),
            pl.BlockSpec((1, 1, H), lambda i: (i, 0, 0)),
        ],
        out_specs=[
            pl.BlockSpec((E, H), lambda i: (0, 0)),
            pl.BlockSpec((E, TB), lambda i: (0, i)),
        ],
        out_shape=[
            jax.ShapeDtypeStruct((E, H), jnp.float32),
            jax.ShapeDtypeStruct((E, B * S), jnp.float32),
        ],
        scratch_shapes=[pltpu.VMEM((E, E, E), jnp.float32)],
    )(xf, xf, gate_W, gb2, gate_b.reshape(E, 1), ew_half, eb_half)

    zero_fill = functools.partial(
        pl.kernel,
        mesh=plsc.VectorSubcoreMesh(core_axis_name="c", subcore_axis_name="s"),
        out_type=jax.ShapeDtypeStruct((B * S, H), jnp.float32),
        scratch_types=[
            pltpu.VMEM((ZROWS, H), jnp.float32),
            pltpu.SemaphoreType.DMA,
        ],
    )(_zero_fill_kernel)
    out_z = zero_fill()

    out = pl.pallas_call(
        _patch_kernel,
        grid=(1,),
        in_specs=[
            pl.BlockSpec((E, H), lambda i: (0, 0)),
            pl.BlockSpec((E, H), lambda i: (0, 0)),
        ],
        out_specs=pl.BlockSpec((E, H), lambda i: (0, 0)),
        out_shape=jax.ShapeDtypeStruct((B * S, H), jnp.float32),
        input_output_aliases={0: 0},
    )(out_z, out8)

    return out.reshape(B, S, H), logits.T


# speedup vs baseline: 1.5872x; 1.3278x over previous
import jax
import jax.numpy as jnp
from jax.experimental import pallas as pl
import jax.experimental.pallas.tpu as pltpu

HIDDEN = 1024
E = 8
T_TOTAL = 8192
TB = 512
NUM_TB = T_TOTAL // TB


def _dot_t(a, b):
    return jax.lax.dot_general(
        a, b, (((1,), (1,)), ((), ())), preferred_element_type=jnp.float32
    )


def _moe_kernel(x_ref, x8_ref, gw_ref, gb_ref, gbt_ref, ew_ref, eb_ref,
                out_ref, logits_ref, c_ref, acc_ref):
    t = pl.program_id(0)

    xb = x_ref[:, :]
    gw = gw_ref[:, :]
    logits_ref[:, :] = _dot_t(gw, xb) + gbt_ref[:, :]

    out_ref[:, :] = jnp.zeros((TB, HIDDEN), jnp.float32)

    x8 = x8_ref[:, :]

    @pl.when(t == 0)
    def _init():
        l8 = _dot_t(x8, gw) + gb_ref[:, :]
        m = jnp.max(l8, axis=-1, keepdims=True)
        p = jnp.exp(l8 - m)
        p = p / jnp.sum(p, axis=-1, keepdims=True)

        iota_e = jax.lax.broadcasted_iota(
            jnp.int32, (E, E), 1).astype(jnp.float32)
        iota_r = jax.lax.broadcasted_iota(
            jnp.int32, (E, E), 0).astype(jnp.float32)

        rank = jnp.zeros((E, E), jnp.float32)
        for e2 in range(E):
            pe2 = p[:, e2:e2 + 1]
            rank = rank + jnp.where(
                (pe2 > p) | ((pe2 == p) & (e2 < iota_e)), 1.0, 0.0)

        sp = jnp.zeros((E, E), jnp.float32)
        for e in range(E):
            sp = sp + jnp.where(rank[:, e:e + 1] == iota_e,
                                p[:, e:e + 1], 0.0)
        sp = sp / jnp.sum(sp, axis=-1, keepdims=True)

        for i in range(E):
            ri = rank[i:i + 1, :]
            c_ref[i] = sp * jnp.where(ri == iota_r, 1.0, 0.0)

        acc_ref[:, :, :] = jnp.zeros((2, E, HIDDEN // 2), jnp.float32)

    i = t // 2
    h = t % 2
    y = _dot_t(x8, ew_ref[0]) + eb_ref[0]

    acc_ref[h] += jax.lax.dot_general(
        c_ref[i], y, (((1,), (0,)), ((), ())),
        preferred_element_type=jnp.float32)

    @pl.when(t == NUM_TB - 1)
    def _final():
        out_ref[0:E, 0:HIDDEN // 2] = acc_ref[0]
        out_ref[0:E, HIDDEN // 2:HIDDEN] = acc_ref[1]


@jax.jit
def kernel(x, gate_W, gate_b, expert_W, expert_b):
    B, S, H = x.shape
    xf = x.reshape(B * S, H)
    gb2 = gate_b.reshape(1, E)
    gbt = gate_b.reshape(E, 1)
    ew_half = expert_W.reshape(2 * E, H // 2, H)
    eb_half = expert_b.reshape(2 * E, 1, H // 2)

    out, logits_t = pl.pallas_call(
        _moe_kernel,
        grid=(NUM_TB,),
        in_specs=[
            pl.BlockSpec((TB, H), lambda i: ((i + 1) % NUM_TB, 0)),
            pl.BlockSpec((E, H), lambda i: (0, 0)),
            pl.BlockSpec((E, H), lambda i: (0, 0)),
            pl.BlockSpec((1, E), lambda i: (0, 0)),
            pl.BlockSpec((E, 1), lambda i: (0, 0)),
            pl.BlockSpec((1, H // 2, H), lambda i: (i, 0, 0)),
            pl.BlockSpec((1, 1, H // 2), lambda i: (i, 0, 0)),
        ],
        out_specs=[
            pl.BlockSpec((TB, H), lambda i: ((i + 1) % NUM_TB, 0)),
            pl.BlockSpec((E, TB), lambda i: (0, (i + 1) % NUM_TB)),
        ],
        out_shape=[
            jax.ShapeDtypeStruct((B * S, H), jnp.float32),
            jax.ShapeDtypeStruct((E, B * S), jnp.float32),
        ],
        scratch_shapes=[
            pltpu.VMEM((E, E, E), jnp.float32),
            pltpu.VMEM((2, E, HIDDEN // 2), jnp.float32),
        ],
    )(xf, xf, gate_W, gb2, gbt, ew_half, eb_half)

    return out.reshape(B, S, H), logits_t.T


# speedup vs baseline: 1.6994x; 1.0707x over previous
import jax
import jax.numpy as jnp
from jax.experimental import pallas as pl
import jax.experimental.pallas.tpu as pltpu

HIDDEN = 1024
E = 8
T_TOTAL = 8192
TB = 1024
NUM_TB = T_TOTAL // TB


def _dot_t(a, b):
    return jax.lax.dot_general(
        a, b, (((1,), (1,)), ((), ())), preferred_element_type=jnp.float32
    )


def _moe_kernel(x_ref, x8_ref, gw_ref, gb_ref, gbt_ref, ew_ref, eb_ref,
                out_ref, logits_ref, c_ref, acc_ref):
    t = pl.program_id(0)

    xb = x_ref[:, :]
    gw = gw_ref[:, :]
    logits_ref[:, :] = _dot_t(gw, xb) + gbt_ref[:, :]

    out_ref[:, :] = jnp.zeros((TB, HIDDEN), jnp.float32)

    x8 = x8_ref[:, :]

    @pl.when(t == 0)
    def _init():
        l8 = _dot_t(x8, gw) + gb_ref[:, :]
        m = jnp.max(l8, axis=-1, keepdims=True)
        p = jnp.exp(l8 - m)
        p = p / jnp.sum(p, axis=-1, keepdims=True)

        iota_e = jax.lax.broadcasted_iota(
            jnp.int32, (E, E), 1).astype(jnp.float32)
        iota_r = jax.lax.broadcasted_iota(
            jnp.int32, (E, E), 0).astype(jnp.float32)

        rank = jnp.zeros((E, E), jnp.float32)
        for e2 in range(E):
            pe2 = p[:, e2:e2 + 1]
            rank = rank + jnp.where(
                (pe2 > p) | ((pe2 == p) & (e2 < iota_e)), 1.0, 0.0)

        sp = jnp.zeros((E, E), jnp.float32)
        for e in range(E):
            sp = sp + jnp.where(rank[:, e:e + 1] == iota_e,
                                p[:, e:e + 1], 0.0)
        sp = sp / jnp.sum(sp, axis=-1, keepdims=True)

        for i in range(E):
            ri = rank[i:i + 1, :]
            c_ref[i] = sp * jnp.where(ri == iota_r, 1.0, 0.0)

        acc_ref[:, :] = jnp.zeros((E, HIDDEN), jnp.float32)

    y = _dot_t(x8, ew_ref[0]) + eb_ref[0]

    acc_ref[:, :] += jax.lax.dot_general(
        c_ref[t], y, (((1,), (0,)), ((), ())),
        preferred_element_type=jnp.float32)

    @pl.when(t == NUM_TB - 1)
    def _final():
        out_ref[0:E, :] = acc_ref[:, :]


@jax.jit
def kernel(x, gate_W, gate_b, expert_W, expert_b):
    B, S, H = x.shape
    xf = x.reshape(B * S, H)
    gb2 = gate_b.reshape(1, E)
    gbt = gate_b.reshape(E, 1)
    ew_half = expert_W
    eb_half = expert_b.reshape(E, 1, H)

    out, logits_t = pl.pallas_call(
        _moe_kernel,
        grid=(NUM_TB,),
        in_specs=[
            pl.BlockSpec((TB, H), lambda i: ((i + 1) % NUM_TB, 0)),
            pl.BlockSpec((E, H), lambda i: (0, 0)),
            pl.BlockSpec((E, H), lambda i: (0, 0)),
            pl.BlockSpec((1, E), lambda i: (0, 0)),
            pl.BlockSpec((E, 1), lambda i: (0, 0)),
            pl.BlockSpec((1, H, H), lambda i: (i, 0, 0)),
            pl.BlockSpec((1, 1, H), lambda i: (i, 0, 0)),
        ],
        out_specs=[
            pl.BlockSpec((TB, H), lambda i: ((i + 1) % NUM_TB, 0)),
            pl.BlockSpec((E, TB), lambda i: (0, (i + 1) % NUM_TB)),
        ],
        out_shape=[
            jax.ShapeDtypeStruct((B * S, H), jnp.float32),
            jax.ShapeDtypeStruct((E, B * S), jnp.float32),
        ],
        scratch_shapes=[
            pltpu.VMEM((E, E, E), jnp.float32),
            pltpu.VMEM((E, HIDDEN), jnp.float32),
        ],
    )(xf, xf, gate_W, gb2, gbt, ew_half, eb_half)

    return out.reshape(B, S, H), logits_t.T
